# baseline (speedup 1.0000x reference)
import functools

import jax
import jax.numpy as jnp
from jax import lax
from jax.experimental import pallas as pl
from jax.experimental.pallas import tpu as pltpu

_ALPHA = 0.3


def _esn_recurrence_kernel(pre_ref, h0_ref, wr_ref,
                           h_seq_ref, h_carry, *, tt):
    c = pl.program_id(1)

    @pl.when(c == 0)
    def _():
        h_carry[...] = h0_ref[...]

    wr = wr_ref[...]
    om_a = jnp.float32(1.0 - _ALPHA)
    a = jnp.float32(_ALPHA)

    def body(s, h):
        pre = pre_ref[s] + jnp.dot(h, wr,
                                   preferred_element_type=jnp.float32)
        h_new = h * om_a + a * jnp.tanh(pre)
        h_seq_ref[s] = h_new
        return h_new

    h_final = lax.fori_loop(0, tt, body, h_carry[...], unroll=True)
    h_carry[...] = h_final


@jax.jit
def _esn_forward(x_seq, h0, win_t, wr_t, wout_t):
    T, B, n_in = x_seq.shape
    R = h0.shape[-1]
    nb = 2
    bb = B // nb
    tt = 8
    nc = T // tt

    pre_in = jnp.dot(x_seq.reshape(T * B, n_in), win_t,
                     preferred_element_type=jnp.float32).reshape(T, B, R)

    h_seq = pl.pallas_call(
        functools.partial(_esn_recurrence_kernel, tt=tt),
        out_shape=jax.ShapeDtypeStruct((T, B, R), jnp.float32),
        grid=(nb, nc),
        in_specs=[
            pl.BlockSpec((tt, bb, R), lambda b, c: (c, b, 0)),
            pl.BlockSpec((bb, R), lambda b, c: (b, 0)),
            pl.BlockSpec((R, R), lambda b, c: (0, 0)),
        ],
        out_specs=pl.BlockSpec((tt, bb, R), lambda b, c: (c, b, 0)),
        scratch_shapes=[
            pltpu.VMEM((bb, R), jnp.float32),
        ],
        compiler_params=pltpu.CompilerParams(
            dimension_semantics=("parallel", "arbitrary")),
    )(pre_in, h0, wr_t)

    out_seq = jnp.dot(h_seq.reshape(T * B, R), wout_t,
                      preferred_element_type=jnp.float32).reshape(T, B, n_in)
    return out_seq, h_seq


def kernel(x_seq, h0, win_t, wr_t, wout_t):
    return _esn_forward(x_seq, h0, win_t, wr_t, wout_t)


# speedup vs baseline: 1.3131x; 1.3131x over previous
import functools

import jax
import jax.numpy as jnp
from jax import lax
from jax.experimental import pallas as pl
from jax.experimental.pallas import tpu as pltpu

_ALPHA = 0.3


def _esn_fused_kernel(x_ref, h0_ref, win_ref, wr_ref, wout_ref,
                      h_seq_ref, out_ref, h_carry, pre_scratch, *, tt):
    c = pl.program_id(1)

    @pl.when(c == 0)
    def _():
        h_carry[...] = h0_ref[...]

    bb, r = h0_ref.shape
    n_in = x_ref.shape[2]

    pre_scratch[...] = jnp.dot(
        x_ref[...].reshape(tt * bb, n_in), win_ref[...],
        preferred_element_type=jnp.float32).reshape(tt, bb, r)

    wr = wr_ref[...]
    om_a = jnp.float32(1.0 - _ALPHA)
    a = jnp.float32(_ALPHA)

    def body(s, h):
        pre = pre_scratch[s] + jnp.dot(h, wr,
                                       preferred_element_type=jnp.float32)
        h_new = h * om_a + a * jnp.tanh(pre)
        h_seq_ref[s] = h_new
        return h_new

    h_final = lax.fori_loop(0, tt, body, h_carry[...], unroll=True)
    h_carry[...] = h_final

    out_ref[...] = jnp.dot(
        h_seq_ref[...].reshape(tt * bb, r).astype(jnp.bfloat16),
        wout_ref[...],
        preferred_element_type=jnp.float32).reshape(tt, bb, n_in)


@jax.jit
def _esn_forward(x_seq, h0, win_t, wr_t, wout_t):
    T, B, n_in = x_seq.shape
    R = h0.shape[-1]
    nb = 2
    bb = B // nb
    tt = 8
    nc = T // tt

    x_bf = x_seq.astype(jnp.bfloat16)
    win_bf = win_t.astype(jnp.bfloat16)
    wout_bf = wout_t.astype(jnp.bfloat16)

    h_seq, out_seq = pl.pallas_call(
        functools.partial(_esn_fused_kernel, tt=tt),
        out_shape=[
            jax.ShapeDtypeStruct((T, B, R), jnp.float32),
            jax.ShapeDtypeStruct((T, B, n_in), jnp.float32),
        ],
        grid=(nb, nc),
        in_specs=[
            pl.BlockSpec((tt, bb, n_in), lambda b, c: (c, b, 0)),
            pl.BlockSpec((bb, R), lambda b, c: (b, 0)),
            pl.BlockSpec((n_in, R), lambda b, c: (0, 0)),
            pl.BlockSpec((R, R), lambda b, c: (0, 0)),
            pl.BlockSpec((R, n_in), lambda b, c: (0, 0)),
        ],
        out_specs=[
            pl.BlockSpec((tt, bb, R), lambda b, c: (c, b, 0)),
            pl.BlockSpec((tt, bb, n_in), lambda b, c: (c, b, 0)),
        ],
        scratch_shapes=[
            pltpu.VMEM((bb, R), jnp.float32),
            pltpu.VMEM((tt, bb, R), jnp.float32),
        ],
        compiler_params=pltpu.CompilerParams(
            dimension_semantics=("parallel", "arbitrary")),
    )(x_bf, h0, win_bf, wr_t, wout_bf)
    return out_seq, h_seq


def kernel(x_seq, h0, win_t, wr_t, wout_t):
    return _esn_forward(x_seq, h0, win_t, wr_t, wout_t)
